# speedup vs baseline: 1.5564x; 1.5564x over previous
import functools

import jax
import jax.numpy as jnp
from jax import lax
from jax.experimental import pallas as pl
from jax.experimental.pallas import tpu as pltpu
from jax.experimental.pallas import tpu_sc as plsc

NC, NS = 2, 16
NW = NC * NS
LN_EPS_ = 1e-3


def _sc_gather(table, idx_flat):
    n_tok = idx_flat.shape[0]
    h = table.shape[1]
    b_per_w = n_tok // NW
    chunk = 64
    n_chunks = b_per_w // chunk

    mesh = plsc.VectorSubcoreMesh(
        core_axis_name="c", subcore_axis_name="s",
        num_cores=NC, num_subcores=NS)

    @functools.partial(
        pl.kernel,
        mesh=mesh,
        out_type=jax.ShapeDtypeStruct((n_tok, h), jnp.float32),
        scratch_types=[
            pltpu.VMEM((chunk,), jnp.int32),
            pltpu.VMEM((chunk, h), jnp.float32),
            pltpu.SemaphoreType.DMA,
        ],
    )
    def gather_kernel(table_hbm, idx_hbm, out_hbm, idx_v, rows_v, sem):
        wid = lax.axis_index("s") * NC + lax.axis_index("c")
        base = wid * b_per_w

        def body(c, carry):
            off = base + c * chunk
            pltpu.sync_copy(idx_hbm.at[pl.ds(off, chunk)], idx_v)
            pltpu.async_copy(table_hbm.at[idx_v], rows_v, sem).wait()
            pltpu.sync_copy(rows_v, out_hbm.at[pl.ds(off, chunk)])
            return carry

        lax.fori_loop(0, n_chunks, body, 0)

    return gather_kernel(table, idx_flat)


def _tc_add_ln(gathered, ttf, type_emb, pos_emb, gamma, beta):
    n_tok, h = gathered.shape
    seq = pos_emb.shape[0]
    t_blk = 512
    grid = n_tok // t_blk
    pos_blocks = seq // t_blk

    def body(x_ref, tt_ref, te_ref, pos_ref, g_ref, b_ref, o_ref):
        x = x_ref[...]
        t0 = te_ref[0:1, :]
        dt = te_ref[1:2, :] - t0
        x = x + pos_ref[...] + t0 + tt_ref[...] * dt
        m = jnp.mean(x, axis=-1, keepdims=True)
        d = x - m
        v = jnp.mean(d * d, axis=-1, keepdims=True)
        o_ref[...] = d * lax.rsqrt(v + LN_EPS_) * g_ref[...] + b_ref[...]

    return pl.pallas_call(
        body,
        grid=(grid,),
        in_specs=[
            pl.BlockSpec((t_blk, h), lambda i: (i, 0)),
            pl.BlockSpec((t_blk, 1), lambda i: (i, 0)),
            pl.BlockSpec((2, h), lambda i: (0, 0)),
            pl.BlockSpec((t_blk, h), lambda i: (i % pos_blocks, 0)),
            pl.BlockSpec((1, h), lambda i: (0, 0)),
            pl.BlockSpec((1, h), lambda i: (0, 0)),
        ],
        out_specs=pl.BlockSpec((t_blk, h), lambda i: (i, 0)),
        out_shape=jax.ShapeDtypeStruct((n_tok, h), jnp.float32),
    )(gathered, ttf, type_emb, pos_emb, gamma, beta)


def kernel(input_ids, token_type_ids, word_embeddings, token_type_embeddings,
           position_embeddings, ln_gamma, ln_beta):
    b, s = input_ids.shape
    h = word_embeddings.shape[1]
    idx_flat = input_ids.reshape(-1).astype(jnp.int32)
    gathered = _sc_gather(word_embeddings, idx_flat)
    ttf = token_type_ids.reshape(-1, 1).astype(jnp.float32)
    pos = lax.dynamic_slice_in_dim(position_embeddings, 0, s, axis=0)
    out = _tc_add_ln(gathered, ttf, type_emb=token_type_embeddings,
                     pos_emb=pos, gamma=ln_gamma.reshape(1, h),
                     beta=ln_beta.reshape(1, h))
    return out.reshape(b, s, h)


# speedup vs baseline: 1.7333x; 1.1137x over previous
import functools

import jax
import jax.numpy as jnp
from jax import lax
from jax.experimental import pallas as pl
from jax.experimental.pallas import tpu as pltpu
from jax.experimental.pallas import tpu_sc as plsc

NC, NS = 2, 16
NW = NC * NS
LN_EPS_ = 1e-3


def _sc_gather(table, idx_flat):
    n_tok = idx_flat.shape[0]
    h = table.shape[1]
    b_per_w = n_tok // NW
    chunk = 64
    n_chunks = b_per_w // chunk

    mesh = plsc.VectorSubcoreMesh(
        core_axis_name="c", subcore_axis_name="s",
        num_cores=NC, num_subcores=NS)

    @functools.partial(
        pl.kernel,
        mesh=mesh,
        out_type=jax.ShapeDtypeStruct((n_tok, h), jnp.float32),
        scratch_types=[
            pltpu.VMEM((b_per_w,), jnp.int32),
            pltpu.VMEM((chunk, h), jnp.float32),
            pltpu.VMEM((chunk, h), jnp.float32),
            pltpu.SemaphoreType.DMA,
            pltpu.SemaphoreType.DMA,
        ],
    )
    def gather_kernel(table_hbm, idx_hbm, out_hbm, idx_v, rows0, rows1,
                      sem0, sem1):
        wid = lax.axis_index("s") * NC + lax.axis_index("c")
        base = wid * b_per_w
        bufs = (rows0, rows1)
        sems = (sem0, sem1)

        pltpu.sync_copy(idx_hbm.at[pl.ds(base, b_per_w)], idx_v)

        def start(c):
            return pltpu.async_copy(
                table_hbm.at[idx_v.at[pl.ds(c * chunk, chunk)]],
                bufs[c % 2], sems[c % 2])

        cp = start(0)
        for c in range(n_chunks):
            cp.wait()
            if c + 1 < n_chunks:
                cp = start(c + 1)
            pltpu.sync_copy(bufs[c % 2],
                            out_hbm.at[pl.ds(base + c * chunk, chunk)])

    return gather_kernel(table, idx_flat)


def _tc_add_ln(gathered, ttf, type_emb, pos_emb, gamma, beta):
    n_tok, h = gathered.shape
    seq = pos_emb.shape[0]
    t_blk = 1024
    grid = n_tok // t_blk

    pos_blk = seq // t_blk if seq >= t_blk else 1

    def body(x_ref, tt_ref, te_ref, pos_ref, g_ref, b_ref, o_ref):
        i = pl.program_id(0)
        x = x_ref[...]
        t0 = te_ref[0:1, :]
        dt = te_ref[1:2, :] - t0
        pos = pos_ref[pl.ds((i % pos_blk) * t_blk, t_blk), :]
        x = x + pos + t0 + tt_ref[...] * dt
        m = jnp.mean(x, axis=-1, keepdims=True)
        d = x - m
        v = jnp.mean(d * d, axis=-1, keepdims=True)
        o_ref[...] = d * lax.rsqrt(v + LN_EPS_) * g_ref[...] + b_ref[...]

    return pl.pallas_call(
        body,
        grid=(grid,),
        in_specs=[
            pl.BlockSpec((t_blk, h), lambda i: (i, 0)),
            pl.BlockSpec((t_blk, 1), lambda i: (i, 0)),
            pl.BlockSpec((2, h), lambda i: (0, 0)),
            pl.BlockSpec((seq, h), lambda i: (0, 0)),
            pl.BlockSpec((1, h), lambda i: (0, 0)),
            pl.BlockSpec((1, h), lambda i: (0, 0)),
        ],
        out_specs=pl.BlockSpec((t_blk, h), lambda i: (i, 0)),
        out_shape=jax.ShapeDtypeStruct((n_tok, h), jnp.float32),
    )(gathered, ttf, type_emb, pos_emb, gamma, beta)


def kernel(input_ids, token_type_ids, word_embeddings, token_type_embeddings,
           position_embeddings, ln_gamma, ln_beta):
    b, s = input_ids.shape
    h = word_embeddings.shape[1]
    idx_flat = input_ids.reshape(-1).astype(jnp.int32)
    gathered = _sc_gather(word_embeddings, idx_flat)
    ttf = token_type_ids.reshape(-1, 1).astype(jnp.float32)
    pos = lax.dynamic_slice_in_dim(position_embeddings, 0, s, axis=0)
    out = _tc_add_ln(gathered, ttf, type_emb=token_type_embeddings,
                     pos_emb=pos, gamma=ln_gamma.reshape(1, h),
                     beta=ln_beta.reshape(1, h))
    return out.reshape(b, s, h)
